# baseline (speedup 1.0000x reference)
import functools

import jax
import jax.numpy as jnp
from jax.experimental import pallas as pl
from jax.experimental.pallas import tpu as pltpu


def _pick_tile_k(k_dim: int) -> int:
    for cand in (1792, 896, 512, 448, 256, 128):
        if k_dim % cand == 0:
            return cand
    return k_dim


def _body(x_ref, w1_ref, b1_ref, w2_ref, b2_ref, wc_ref, bc_ref, wb_ref,
          bb_ref, score_ref, bbox_ref, acc_ref, *, nk):
    k = pl.program_id(1)

    @pl.when(k == 0)
    def _init():
        acc_ref[...] = jnp.zeros_like(acc_ref)

    acc_ref[...] += jnp.dot(x_ref[...], w1_ref[...],
                            preferred_element_type=jnp.float32)

    @pl.when(k == nk - 1)
    def _finish():
        h = jnp.maximum(acc_ref[...] + b1_ref[...], 0.0)
        h = jnp.maximum(
            jnp.dot(h, w2_ref[...], preferred_element_type=jnp.float32)
            + b2_ref[...], 0.0)
        score_ref[...] = (
            jnp.dot(h, wc_ref[...], preferred_element_type=jnp.float32)
            + bc_ref[...])
        bbox_ref[...] = (
            jnp.dot(h, wb_ref[...], preferred_element_type=jnp.float32)
            + bb_ref[...])


def kernel(x, W1, b1, W2, b2, Wc, bc, Wb, bb):
    n, k_dim = x.shape
    mid = W1.shape[1]
    nc = Wc.shape[1]
    nb = Wb.shape[1]

    tile_n = min(1024, n)
    tile_k = _pick_tile_k(k_dim)
    nt = pl.cdiv(n, tile_n)
    nk = k_dim // tile_k

    b1_2 = b1.reshape(1, -1)
    b2_2 = b2.reshape(1, -1)
    bc_2 = bc.reshape(1, -1)
    bb_2 = bb.reshape(1, -1)

    grid = (nt, nk)
    out_shapes = (
        jax.ShapeDtypeStruct((n, nc), jnp.float32),
        jax.ShapeDtypeStruct((n, nb), jnp.float32),
    )
    in_specs = [
        pl.BlockSpec((tile_n, tile_k), lambda i, k: (i, k)),
        pl.BlockSpec((tile_k, mid), lambda i, k: (k, 0)),
        pl.BlockSpec((1, mid), lambda i, k: (0, 0)),
        pl.BlockSpec((mid, mid), lambda i, k: (0, 0)),
        pl.BlockSpec((1, mid), lambda i, k: (0, 0)),
        pl.BlockSpec((mid, nc), lambda i, k: (0, 0)),
        pl.BlockSpec((1, nc), lambda i, k: (0, 0)),
        pl.BlockSpec((mid, nb), lambda i, k: (0, 0)),
        pl.BlockSpec((1, nb), lambda i, k: (0, 0)),
    ]
    out_specs = (
        pl.BlockSpec((tile_n, nc), lambda i, k: (i, 0)),
        pl.BlockSpec((tile_n, nb), lambda i, k: (i, 0)),
    )

    return pl.pallas_call(
        functools.partial(_body, nk=nk),
        grid=grid,
        in_specs=in_specs,
        out_specs=out_specs,
        out_shape=out_shapes,
        scratch_shapes=[pltpu.VMEM((tile_n, mid), jnp.float32)],
        compiler_params=pltpu.CompilerParams(
            dimension_semantics=("parallel", "arbitrary"),
        ),
    )(x, W1, b1_2, W2, b2_2, Wc, bc_2, Wb, bb_2)
